# Initial kernel scaffold; baseline (speedup 1.0000x reference)
import jax
import jax.numpy as jnp
from jax.experimental import pallas as pl


def kernel(img, spx):
    raise NotImplementedError("write your pallas kernel here")


# speedup vs baseline: 1.1789x; 1.1789x over previous
import functools

import jax
import jax.numpy as jnp
from jax import lax
from jax.experimental import pallas as pl
from jax.experimental.pallas import tpu as pltpu
from jax.experimental.pallas import tpu_sc as plsc

L = 16
NC = 2
NS = 16
NW = NC * NS
K = 1024


def _pool_body(img_hbm, spx_hbm, partial_hbm, idx_v, img_v, acc_v, red_v):
    B, C, NPIX = img_hbm.shape
    ranges_per_batch = NW // B
    npix_t = NPIX // ranges_per_batch

    cid = lax.axis_index("c")
    sid = lax.axis_index("s")
    wid = sid * NC + cid
    b = wid // ranges_per_batch
    r = wid % ranges_per_batch
    base = r * npix_t

    neg = jnp.full((L,), -jnp.inf, jnp.float32)
    lane_off = lax.iota(jnp.int32, L) * K

    pltpu.sync_copy(spx_hbm.at[b, pl.ds(base, npix_t)], idx_v)

    def _init(i, carry):
        acc_v[pl.ds(i * L, L)] = neg
        return carry

    lax.fori_loop(0, L * K // L, _init, 0)

    def _flatten(i, carry):
        idx_v[pl.ds(i * L, L)] = idx_v[pl.ds(i * L, L)] + lane_off
        return carry

    lax.fori_loop(0, npix_t // L, _flatten, 0)

    def _channel(c, carry):
        pltpu.sync_copy(img_hbm.at[b, c, pl.ds(base, npix_t)], img_v)

        def _update(i, carry2):
            fidx = idx_v[pl.ds(i * L, L)]
            val = img_v[pl.ds(i * L, L)]
            old = plsc.load_gather(acc_v, [fidx])
            plsc.store_scatter(acc_v, [fidx], jnp.maximum(old, val))
            return carry2

        lax.fori_loop(0, npix_t // L, _update, 0)

        def _reduce(g, carry2):
            m = acc_v[pl.ds(g * L, L)]
            acc_v[pl.ds(g * L, L)] = neg
            for l in range(1, L):
                off = l * K + g * L
                m = jnp.maximum(m, acc_v[pl.ds(off, L)])
                acc_v[pl.ds(off, L)] = neg
            red_v[pl.ds(g * L, L)] = m
            return carry2

        lax.fori_loop(0, K // L, _reduce, 0)
        pltpu.sync_copy(red_v, partial_hbm.at[b, r, c])
        return carry

    lax.fori_loop(0, C, _channel, 0)


def _merge_body(p_ref, o_ref):
    o_ref[...] = jnp.max(p_ref[...], axis=1)


@jax.jit
def kernel(img, spx):
    B, C, H, W = img.shape
    npix = H * W
    img3 = img.reshape(B, C, npix)
    spx2 = spx.reshape(B, npix)
    ranges_per_batch = NW // B

    mesh = plsc.VectorSubcoreMesh(
        core_axis_name="c", subcore_axis_name="s", num_cores=NC,
        num_subcores=NS)
    npix_t = npix // ranges_per_batch
    pool = pl.kernel(
        _pool_body,
        out_type=jax.ShapeDtypeStruct((B, ranges_per_batch, C, K),
                                      jnp.float32),
        mesh=mesh,
        compiler_params=pltpu.CompilerParams(needs_layout_passes=False),
        scratch_types=[
            pltpu.VMEM((npix_t,), jnp.int32),
            pltpu.VMEM((npix_t,), jnp.float32),
            pltpu.VMEM((L * K,), jnp.float32),
            pltpu.VMEM((K,), jnp.float32),
        ],
    )
    partial = pool(img3, spx2)

    out = pl.pallas_call(
        _merge_body,
        grid=(B,),
        in_specs=[pl.BlockSpec((1, ranges_per_batch, C, K),
                               lambda i: (i, 0, 0, 0))],
        out_specs=pl.BlockSpec((1, C, K), lambda i: (i, 0, 0)),
        out_shape=jax.ShapeDtypeStruct((B, C, K), jnp.float32),
    )(partial)
    return out


# speedup vs baseline: 1.2799x; 1.0857x over previous
import functools

import jax
import jax.numpy as jnp
from jax import lax
from jax.experimental import pallas as pl
from jax.experimental.pallas import tpu as pltpu
from jax.experimental.pallas import tpu_sc as plsc

L = 16
NC = 2
NS = 16
NW = NC * NS
K = 1024


def _pool_body(img_hbm, spx_hbm, partial_hbm, idx_v, img_v, acc_v, red_v):
    B, C, NPIX = img_hbm.shape
    ranges_per_batch = NW // B
    npix_t = NPIX // ranges_per_batch

    cid = lax.axis_index("c")
    sid = lax.axis_index("s")
    wid = sid * NC + cid
    b = wid // ranges_per_batch
    r = wid % ranges_per_batch
    base = r * npix_t

    neg = jnp.full((L,), -jnp.inf, jnp.float32)
    lane_off = lax.iota(jnp.int32, L) * K

    pltpu.sync_copy(spx_hbm.at[b, pl.ds(base, npix_t)], idx_v)

    @plsc.parallel_loop(0, L * K // L, unroll=8)
    def _init(i):
        acc_v[pl.ds(i * L, L)] = neg

    @plsc.parallel_loop(0, npix_t // L, unroll=8)
    def _flatten(i):
        idx_v[pl.ds(i * L, L)] = idx_v[pl.ds(i * L, L)] + lane_off

    UNROLL = 8

    def _channel(c, carry):
        pltpu.sync_copy(img_hbm.at[b, c, pl.ds(base, npix_t)], img_v)

        def _update(i, carry2):
            for u in range(UNROLL):
                off = (i * UNROLL + u) * L
                fidx = idx_v[pl.ds(off, L)]
                val = img_v[pl.ds(off, L)]
                old = plsc.load_gather(acc_v, [fidx])
                plsc.store_scatter(acc_v, [fidx], jnp.maximum(old, val))
            return carry2

        lax.fori_loop(0, npix_t // L // UNROLL, _update, 0)

        @plsc.parallel_loop(0, K // L, unroll=2)
        def _reduce(g):
            m = acc_v[pl.ds(g * L, L)]
            acc_v[pl.ds(g * L, L)] = neg
            for l in range(1, L):
                off = l * K + g * L
                m = jnp.maximum(m, acc_v[pl.ds(off, L)])
                acc_v[pl.ds(off, L)] = neg
            red_v[pl.ds(g * L, L)] = m

        pltpu.sync_copy(red_v, partial_hbm.at[b, r, c])
        return carry

    lax.fori_loop(0, C, _channel, 0)


def _merge_body(p_ref, o_ref):
    o_ref[...] = jnp.max(p_ref[...], axis=1)


@jax.jit
def kernel(img, spx):
    B, C, H, W = img.shape
    npix = H * W
    img3 = img.reshape(B, C, npix)
    spx2 = spx.reshape(B, npix)
    ranges_per_batch = NW // B

    mesh = plsc.VectorSubcoreMesh(
        core_axis_name="c", subcore_axis_name="s", num_cores=NC,
        num_subcores=NS)
    npix_t = npix // ranges_per_batch
    pool = pl.kernel(
        _pool_body,
        out_type=jax.ShapeDtypeStruct((B, ranges_per_batch, C, K),
                                      jnp.float32),
        mesh=mesh,
        compiler_params=pltpu.CompilerParams(needs_layout_passes=False),
        scratch_types=[
            pltpu.VMEM((npix_t,), jnp.int32),
            pltpu.VMEM((npix_t,), jnp.float32),
            pltpu.VMEM((L * K,), jnp.float32),
            pltpu.VMEM((K,), jnp.float32),
        ],
    )
    partial = pool(img3, spx2)

    out = pl.pallas_call(
        _merge_body,
        grid=(B,),
        in_specs=[pl.BlockSpec((1, ranges_per_batch, C, K),
                               lambda i: (i, 0, 0, 0))],
        out_specs=pl.BlockSpec((1, C, K), lambda i: (i, 0, 0)),
        out_shape=jax.ShapeDtypeStruct((B, C, K), jnp.float32),
    )(partial)
    return out


# speedup vs baseline: 1.6714x; 1.3058x over previous
import functools

import jax
import jax.numpy as jnp
from jax import lax
from jax.experimental import pallas as pl
from jax.experimental.pallas import tpu as pltpu
from jax.experimental.pallas import tpu_sc as plsc

L = 16
NC = 2
NS = 16
NW = NC * NS
K = 1024
NQ = 4
UNROLL = 4


def _pool_body(img_hbm, spx_hbm, partial_hbm, idx_v, ia0, ia1, ib0, ib1,
               acc_a, acc_b, red_a, red_b, sa0, sa1, sb0, sb1):
    B, C, NPIX = img_hbm.shape
    ranges_per_batch = NW // B
    npix_t = NPIX // ranges_per_batch
    q_pix = npix_t // NQ

    cid = lax.axis_index("c")
    sid = lax.axis_index("s")
    wid = sid * NC + cid
    b = wid // ranges_per_batch
    r = wid % ranges_per_batch
    base = r * npix_t

    img_bufs = (ia0, ia1), (ib0, ib1)
    sems = (sa0, sa1), (sb0, sb1)

    neg = jnp.full((L,), -jnp.inf, jnp.float32)
    lane_off = lax.iota(jnp.int32, L) * K

    pltpu.sync_copy(spx_hbm.at[b, pl.ds(base, npix_t)], idx_v)

    @plsc.parallel_loop(0, L * K // L, unroll=8)
    def _init(i):
        acc_a[pl.ds(i * L, L)] = neg
        acc_b[pl.ds(i * L, L)] = neg

    @plsc.parallel_loop(0, npix_t // L, unroll=8)
    def _flatten(i):
        idx_v[pl.ds(i * L, L)] = idx_v[pl.ds(i * L, L)] + lane_off

    def _start_q(c0, q, buf_par):
        for ch in range(2):
            pltpu.async_copy(
                img_hbm.at[b, c0 + ch, pl.ds(base + q * q_pix, q_pix)],
                img_bufs[ch][buf_par], sems[ch][buf_par])

    def _wait_q(c0, q, buf_par):
        for ch in range(2):
            pltpu.make_async_copy(
                img_hbm.at[b, c0 + ch, pl.ds(base + q * q_pix, q_pix)],
                img_bufs[ch][buf_par], sems[ch][buf_par]).wait()

    _start_q(0, 0, 0)

    def _pair(p, carry):
        c0 = 2 * p
        for q in range(NQ):
            par = q % 2
            _wait_q(c0, q, par)
            if q + 1 < NQ:
                _start_q(c0, q + 1, (q + 1) % 2)
            if q == NQ - 1:
                @pl.when(p + 1 < C // 2)
                def _():
                    _start_q(c0 + 2, 0, 0)
            ia, ib = img_bufs[0][par], img_bufs[1][par]

            def _update(i, carry2):
                for u in range(UNROLL):
                    off = (i * UNROLL + u) * L
                    fidx = idx_v[pl.ds(q * q_pix + off, L)]
                    va = ia[pl.ds(off, L)]
                    vb = ib[pl.ds(off, L)]
                    oa = plsc.load_gather(acc_a, [fidx])
                    plsc.store_scatter(acc_a, [fidx], jnp.maximum(oa, va))
                    ob = plsc.load_gather(acc_b, [fidx])
                    plsc.store_scatter(acc_b, [fidx], jnp.maximum(ob, vb))
                return carry2

            lax.fori_loop(0, q_pix // L // UNROLL, _update, 0)

        @plsc.parallel_loop(0, K // L, unroll=2)
        def _reduce(g):
            ma = acc_a[pl.ds(g * L, L)]
            mb = acc_b[pl.ds(g * L, L)]
            acc_a[pl.ds(g * L, L)] = neg
            acc_b[pl.ds(g * L, L)] = neg
            for l in range(1, L):
                off = l * K + g * L
                ma = jnp.maximum(ma, acc_a[pl.ds(off, L)])
                mb = jnp.maximum(mb, acc_b[pl.ds(off, L)])
                acc_a[pl.ds(off, L)] = neg
                acc_b[pl.ds(off, L)] = neg
            red_a[pl.ds(g * L, L)] = ma
            red_b[pl.ds(g * L, L)] = mb

        pltpu.sync_copy(red_a, partial_hbm.at[b, r, c0])
        pltpu.sync_copy(red_b, partial_hbm.at[b, r, c0 + 1])
        return carry

    lax.fori_loop(0, C // 2, _pair, 0)


def _merge_body(p_ref, o_ref):
    o_ref[...] = jnp.max(p_ref[...], axis=1)


@jax.jit
def kernel(img, spx):
    B, C, H, W = img.shape
    npix = H * W
    img3 = img.reshape(B, C, npix)
    spx2 = spx.reshape(B, npix)
    ranges_per_batch = NW // B

    mesh = plsc.VectorSubcoreMesh(
        core_axis_name="c", subcore_axis_name="s", num_cores=NC,
        num_subcores=NS)
    npix_t = npix // ranges_per_batch
    q_pix = npix_t // NQ
    pool = pl.kernel(
        _pool_body,
        out_type=jax.ShapeDtypeStruct((B, ranges_per_batch, C, K),
                                      jnp.float32),
        mesh=mesh,
        compiler_params=pltpu.CompilerParams(needs_layout_passes=False),
        scratch_types=[
            pltpu.VMEM((npix_t,), jnp.int32),
            pltpu.VMEM((q_pix,), jnp.float32),
            pltpu.VMEM((q_pix,), jnp.float32),
            pltpu.VMEM((q_pix,), jnp.float32),
            pltpu.VMEM((q_pix,), jnp.float32),
            pltpu.VMEM((L * K,), jnp.float32),
            pltpu.VMEM((L * K,), jnp.float32),
            pltpu.VMEM((K,), jnp.float32),
            pltpu.VMEM((K,), jnp.float32),
            pltpu.SemaphoreType.DMA,
            pltpu.SemaphoreType.DMA,
            pltpu.SemaphoreType.DMA,
            pltpu.SemaphoreType.DMA,
        ],
    )
    partial = pool(img3, spx2)

    out = pl.pallas_call(
        _merge_body,
        grid=(B,),
        in_specs=[pl.BlockSpec((1, ranges_per_batch, C, K),
                               lambda i: (i, 0, 0, 0))],
        out_specs=pl.BlockSpec((1, C, K), lambda i: (i, 0, 0)),
        out_shape=jax.ShapeDtypeStruct((B, C, K), jnp.float32),
    )(partial)
    return out
